# baseline (speedup 1.0000x reference)
import functools

import jax
import jax.numpy as jnp
from jax import lax
from jax.experimental import pallas as pl
from jax.experimental.pallas import tpu as pltpu
from jax.experimental.pallas import tpu_sc as plsc

BATCH = 4096
FIELDS = 26
VOCAB = 100000
DIM = 32

NUM_CORES = 2
NUM_SUBCORES = 16
LANES = 16
NW = NUM_CORES * NUM_SUBCORES
PER_TILE = BATCH * FIELDS // NW
CHUNK = 128
CHUNKS = PER_TILE // CHUNK
GROUPS = CHUNK // LANES

_mesh = plsc.VectorSubcoreMesh(core_axis_name="c", subcore_axis_name="s")


@functools.partial(
    pl.kernel,
    mesh=_mesh,
    out_type=jax.ShapeDtypeStruct((NW, CHUNKS, CHUNK, DIM), jnp.float32),
    scratch_types=[
        pltpu.VMEM((CHUNKS, CHUNK), jnp.int32),
        pltpu.VMEM((CHUNKS, CHUNK, DIM), jnp.float32),
        pltpu.SemaphoreType.DMA,
    ],
    compiler_params=pltpu.CompilerParams(use_tc_tiling_on_sc=False),
)
def _gather_kernel(idx_hbm, tab_hbm, out_hbm, idx_v, rows_v, sem):
    wid = lax.axis_index("s") * NUM_CORES + lax.axis_index("c")
    pltpu.sync_copy(idx_hbm.at[wid], idx_v)

    lanes = lax.iota(jnp.int32, LANES)
    copies = []
    for j in range(CHUNKS):
        for g in range(GROUPS):
            c0 = g * LANES
            p0 = (j * CHUNK + c0) % FIELDS
            field = lax.rem(lanes + p0, FIELDS)
            idx_v[j, pl.ds(c0, LANES)] = (
                idx_v[j, pl.ds(c0, LANES)] + field * VOCAB
            )
        cp = pltpu.make_async_copy(tab_hbm.at[idx_v.at[j]], rows_v.at[j], sem)
        cp.start()
        copies.append(cp)
    for cp in copies:
        cp.wait()
    pltpu.sync_copy(rows_v, out_hbm.at[wid])


def kernel(indices, tables):
    idx3 = indices.astype(jnp.int32).reshape(NW, CHUNKS, CHUNK)
    tab2 = tables.reshape(FIELDS * VOCAB, DIM)
    out4 = _gather_kernel(idx3, tab2)
    return out4.reshape(BATCH, FIELDS * DIM)


# speedup vs baseline: 5.3250x; 5.3250x over previous
import functools

import jax
import jax.numpy as jnp
from jax import lax
from jax.experimental import pallas as pl
from jax.experimental.pallas import tpu as pltpu
from jax.experimental.pallas import tpu_sc as plsc

BATCH = 4096
FIELDS = 26
VOCAB = 100000
DIM = 32

NUM_CORES = 2
NUM_SUBCORES = 16
LANES = 16
NW = NUM_CORES * NUM_SUBCORES
UNITS = FIELDS * DIM
UNITS_PER_W = UNITS // NW

_mesh = plsc.VectorSubcoreMesh(core_axis_name="c", subcore_axis_name="s")


@functools.partial(
    pl.kernel,
    mesh=_mesh,
    out_type=jax.ShapeDtypeStruct((UNITS, BATCH), jnp.float32),
    scratch_types=[
        pltpu.VMEM((VOCAB,), jnp.float32),
        pltpu.VMEM((BATCH,), jnp.int32),
        pltpu.VMEM((BATCH,), jnp.float32),
    ],
    compiler_params=pltpu.CompilerParams(
        use_tc_tiling_on_sc=True, needs_layout_passes=False
    ),
)
def _gather_kernel(idx_hbm, tab_hbm, out_hbm, row_v, idx_v, o_v):
    wid = lax.axis_index("s") * NUM_CORES + lax.axis_index("c")

    @pl.loop(0, UNITS_PER_W)
    def _unit(k):
        u = wid * UNITS_PER_W + k
        f = u // DIM
        e = u % DIM
        pltpu.sync_copy(tab_hbm.at[f, e], row_v)
        pltpu.sync_copy(idx_hbm.at[f], idx_v)

        @pl.loop(0, BATCH // LANES, unroll=8)
        def _g(g):
            sl = pl.ds(g * LANES, LANES)
            o_v[sl] = plsc.load_gather(row_v, [idx_v[sl]])

        pltpu.sync_copy(o_v, out_hbm.at[u])


def kernel(indices, tables):
    idx_t = indices.astype(jnp.int32).T
    tab_t = jnp.transpose(tables, (0, 2, 1))
    out = _gather_kernel(idx_t, tab_t)
    return out.T.reshape(BATCH, FIELDS * DIM)


# speedup vs baseline: 5.8323x; 1.0953x over previous
import functools

import jax
import jax.numpy as jnp
from jax import lax
from jax.experimental import pallas as pl
from jax.experimental.pallas import tpu as pltpu
from jax.experimental.pallas import tpu_sc as plsc

BATCH = 4096
FIELDS = 26
VOCAB = 100000
DIM = 32

NUM_CORES = 2
NUM_SUBCORES = 16
LANES = 16
NW = NUM_CORES * NUM_SUBCORES
UNITS = FIELDS * DIM
UNITS_PER_W = UNITS // NW

VMAIN = (VOCAB // 128) * 128
VPAD = VMAIN + 128
BOUNDS = [0, 25088, 50176, 75264, VMAIN]

_mesh = plsc.VectorSubcoreMesh(core_axis_name="c", subcore_axis_name="s")


@functools.partial(
    pl.kernel,
    mesh=_mesh,
    out_type=jax.ShapeDtypeStruct((UNITS, BATCH), jnp.float32),
    scratch_types=[
        pltpu.VMEM((VPAD,), jnp.float32),
        pltpu.VMEM((BATCH,), jnp.int32),
        pltpu.VMEM((BATCH,), jnp.float32),
        pltpu.SemaphoreType.DMA,
    ],
    compiler_params=pltpu.CompilerParams(
        use_tc_tiling_on_sc=True, needs_layout_passes=False
    ),
)
def _gather_kernel(idx_hbm, tab_hbm, tail_hbm, out_hbm, row_v, idx_v, o_v, sem):
    wid = lax.axis_index("s") * NUM_CORES + lax.axis_index("c")

    @pl.loop(0, UNITS_PER_W)
    def _unit(k):
        u = wid * UNITS_PER_W + k
        f = u // DIM
        e = u % DIM
        copies = []
        for s in range(4):
            st, ln = BOUNDS[s], BOUNDS[s + 1] - BOUNDS[s]
            cp = pltpu.make_async_copy(
                tab_hbm.at[f, e, pl.ds(st, ln)], row_v.at[pl.ds(st, ln)], sem
            )
            cp.start()
            copies.append(cp)
        cp = pltpu.make_async_copy(
            tail_hbm.at[f, e], row_v.at[pl.ds(VMAIN, 128)], sem
        )
        cp.start()
        copies.append(cp)

        @pl.when((k == 0) | (e == 0))
        def _():
            pltpu.sync_copy(idx_hbm.at[f], idx_v)

        for cp in copies:
            cp.wait()

        @pl.loop(0, BATCH // LANES, unroll=8)
        def _g(g):
            sl = pl.ds(g * LANES, LANES)
            o_v[sl] = plsc.load_gather(row_v, [idx_v[sl]])

        pltpu.sync_copy(o_v, out_hbm.at[u])


def kernel(indices, tables):
    idx_t = indices.astype(jnp.int32).T
    tab_t = jnp.transpose(tables, (0, 2, 1))
    tail = jnp.pad(tab_t[:, :, VMAIN:], ((0, 0), (0, 0), (0, 96)))
    out = _gather_kernel(idx_t, tab_t, tail)
    return out.T.reshape(BATCH, FIELDS * DIM)


# speedup vs baseline: 5.8556x; 1.0040x over previous
import functools

import jax
import jax.numpy as jnp
from jax import lax
from jax.experimental import pallas as pl
from jax.experimental.pallas import tpu as pltpu
from jax.experimental.pallas import tpu_sc as plsc

BATCH = 4096
FIELDS = 26
VOCAB = 100000
DIM = 32

NUM_CORES = 2
NUM_SUBCORES = 16
LANES = 16
NW = NUM_CORES * NUM_SUBCORES
UNITS = FIELDS * DIM
UNITS_PER_W = UNITS // NW

VMAIN = (VOCAB // 128) * 128
VPAD = VMAIN + 128
BOUNDS = [0, 12544, 25088, 37632, 50176, 62720, 75264, 87808, VMAIN]
NSPLIT = len(BOUNDS) - 1

_mesh = plsc.VectorSubcoreMesh(core_axis_name="c", subcore_axis_name="s")


@functools.partial(
    pl.kernel,
    mesh=_mesh,
    out_type=jax.ShapeDtypeStruct((UNITS, BATCH), jnp.float32),
    scratch_types=[
        pltpu.VMEM((VPAD,), jnp.float32),
        pltpu.VMEM((BATCH,), jnp.int32),
        pltpu.VMEM((2, BATCH), jnp.float32),
        pltpu.SemaphoreType.DMA,
        pltpu.SemaphoreType.DMA,
    ],
    compiler_params=pltpu.CompilerParams(
        use_tc_tiling_on_sc=True, needs_layout_passes=False
    ),
)
def _gather_kernel(
    idx_hbm, tab_hbm, tail_hbm, out_hbm, row_v, idx_v, o_v, sem, osem
):
    wid = lax.axis_index("s") * NUM_CORES + lax.axis_index("c")

    @pl.loop(0, UNITS_PER_W)
    def _unit(k):
        u = wid * UNITS_PER_W + k
        f = u // DIM
        e = u % DIM
        b = k % 2
        copies = []
        for s in range(NSPLIT):
            st, ln = BOUNDS[s], BOUNDS[s + 1] - BOUNDS[s]
            cp = pltpu.make_async_copy(
                tab_hbm.at[f, e, pl.ds(st, ln)], row_v.at[pl.ds(st, ln)], sem
            )
            cp.start()
            copies.append(cp)
        cp = pltpu.make_async_copy(
            tail_hbm.at[f, e], row_v.at[pl.ds(VMAIN, 128)], sem
        )
        cp.start()
        copies.append(cp)

        @pl.when((k == 0) | (e == 0))
        def _():
            pltpu.sync_copy(idx_hbm.at[f], idx_v)

        @pl.when(k >= 2)
        def _():
            pltpu.make_async_copy(o_v.at[b], out_hbm.at[u], osem).wait()

        for cp in copies:
            cp.wait()

        @pl.loop(0, BATCH // LANES, unroll=8)
        def _g(g):
            sl = pl.ds(g * LANES, LANES)
            o_v[b, sl] = plsc.load_gather(row_v, [idx_v[sl]])

        pltpu.make_async_copy(o_v.at[b], out_hbm.at[u], osem).start()

    for t in range(2):
        pltpu.make_async_copy(
            o_v.at[t], out_hbm.at[wid * UNITS_PER_W + t], osem
        ).wait()


def kernel(indices, tables):
    idx_t = indices.astype(jnp.int32).T
    tab_t = jnp.transpose(tables, (0, 2, 1))
    tail = jnp.pad(tab_t[:, :, VMAIN:], ((0, 0), (0, 0), (0, 96)))
    out = _gather_kernel(idx_t, tab_t, tail)
    return out.T.reshape(BATCH, FIELDS * DIM)


# speedup vs baseline: 6.4285x; 1.0978x over previous
import functools

import jax
import jax.numpy as jnp
from jax import lax
from jax.experimental import pallas as pl
from jax.experimental.pallas import tpu as pltpu
from jax.experimental.pallas import tpu_sc as plsc

BATCH = 4096
FIELDS = 26
VOCAB = 100000
DIM = 32

NUM_CORES = 2
NUM_SUBCORES = 16
LANES = 16
NW = NUM_CORES * NUM_SUBCORES
UNITS = FIELDS * DIM
UNITS_PER_W = UNITS // NW

VMAIN = (VOCAB // 128) * 128
HALF = 50048
BOUNDS0 = [0, 12544, 25088, 37632, HALF]
BOUNDS1 = [HALF, 62592, 75136, 87680, VMAIN]
B1LEN = VMAIN - HALF
HBUF = B1LEN + 128

_mesh = plsc.VectorSubcoreMesh(core_axis_name="c", subcore_axis_name="s")


@functools.partial(
    pl.kernel,
    mesh=_mesh,
    out_type=jax.ShapeDtypeStruct((UNITS, BATCH), jnp.float32),
    scratch_types=[
        pltpu.VMEM((HALF,), jnp.float32),
        pltpu.VMEM((HBUF,), jnp.float32),
        pltpu.VMEM((BATCH,), jnp.int32),
        pltpu.VMEM((2, BATCH), jnp.float32),
        pltpu.SemaphoreType.DMA,
        pltpu.SemaphoreType.DMA,
        pltpu.SemaphoreType.DMA,
    ],
    compiler_params=pltpu.CompilerParams(
        use_tc_tiling_on_sc=True, needs_layout_passes=False
    ),
)
def _gather_kernel(
    idx_hbm, tab_hbm, tail_hbm, out_hbm, h0_v, h1_v, idx_v, o_v,
    sem0, sem1, osem,
):
    wid = lax.axis_index("s") * NUM_CORES + lax.axis_index("c")

    def half0_copies(u):
        f = u // DIM
        e = u % DIM
        return [
            pltpu.make_async_copy(
                tab_hbm.at[f, e, pl.ds(st, en - st)],
                h0_v.at[pl.ds(st, en - st)],
                sem0,
            )
            for st, en in zip(BOUNDS0[:-1], BOUNDS0[1:])
        ]

    def half1_copies(u):
        f = u // DIM
        e = u % DIM
        cps = [
            pltpu.make_async_copy(
                tab_hbm.at[f, e, pl.ds(st, en - st)],
                h1_v.at[pl.ds(st - HALF, en - st)],
                sem1,
            )
            for st, en in zip(BOUNDS1[:-1], BOUNDS1[1:])
        ]
        cps.append(
            pltpu.make_async_copy(
                tail_hbm.at[f, e], h1_v.at[pl.ds(B1LEN, 128)], sem1
            )
        )
        return cps

    u0 = wid * UNITS_PER_W
    for cp in half0_copies(u0):
        cp.start()
    for cp in half1_copies(u0):
        cp.start()

    @pl.loop(0, UNITS_PER_W)
    def _unit(k):
        u = u0 + k
        e = u % DIM
        b = k % 2

        @pl.when((k == 0) | (e == 0))
        def _():
            pltpu.sync_copy(idx_hbm.at[u // DIM], idx_v)

        @pl.when(k >= 2)
        def _():
            pltpu.make_async_copy(o_v.at[b], out_hbm.at[u], osem).wait()

        for cp in half0_copies(u):
            cp.wait()

        @pl.loop(0, BATCH // LANES, unroll=8)
        def _g0(g):
            sl = pl.ds(g * LANES, LANES)
            iv = idx_v[sl]
            o_v[b, sl] = plsc.load_gather(h0_v, [jnp.minimum(iv, HALF - 1)])

        @pl.when(k + 1 < UNITS_PER_W)
        def _():
            for cp in half0_copies(u + 1):
                cp.start()

        for cp in half1_copies(u):
            cp.wait()

        @pl.loop(0, BATCH // LANES, unroll=8)
        def _g1(g):
            sl = pl.ds(g * LANES, LANES)
            iv = idx_v[sl]
            hi = plsc.load_gather(
                h1_v, [jnp.maximum(iv - HALF, 0)]
            )
            o_v[b, sl] = jnp.where(iv >= HALF, hi, o_v[b, sl])

        @pl.when(k + 1 < UNITS_PER_W)
        def _():
            for cp in half1_copies(u + 1):
                cp.start()

        pltpu.make_async_copy(o_v.at[b], out_hbm.at[u], osem).start()

    for t in range(2):
        pltpu.make_async_copy(
            o_v.at[t], out_hbm.at[u0 + t], osem
        ).wait()


def kernel(indices, tables):
    idx_t = indices.astype(jnp.int32).T
    tab_t = jnp.transpose(tables, (0, 2, 1))
    tail = jnp.pad(tab_t[:, :, VMAIN:], ((0, 0), (0, 0), (0, 96)))
    out = _gather_kernel(idx_t, tab_t, tail)
    return out.T.reshape(BATCH, FIELDS * DIM)
